# baseline (speedup 1.0000x reference)
import functools

import jax
import jax.numpy as jnp
from jax import lax
from jax.experimental import pallas as pl
from jax.experimental.pallas import tpu as pltpu
from jax.experimental.pallas import tpu_sc as plsc

N_TOK = 2048
N_E = 1024
E_DIM = 256
ALPHA = 0.9

R_TC = 1536
NS_ROWS = N_TOK - R_TC
BI = 512
NB = R_TC // BI
NC = E_DIM // 8

RPT = 16
NPASS = 4
CPP = N_E // NPASS
NJG = CPP // 16


def _tc_body(xt_ref, wt_ref, j_ref, part_ref, acc_ref):
    b = pl.program_id(0)
    c = pl.program_id(1)

    xc = xt_ref[...]
    wc = wt_ref[...]
    xcT = xc.T

    @pl.when(c == 0)
    def _():
        acc_ref[...] = jnp.zeros((BI, N_E), jnp.float32)

    for s in range(BI // 8):
        xs = xcT[8 * s:8 * s + 8, :]

        def sq(k):
            d = xs[:, k:k + 1] - wc[k:k + 1, :]
            return d * d

        g = ((sq(0) + sq(4)) + (sq(2) + sq(6))) + \
            ((sq(1) + sq(5)) + (sq(3) + sq(7)))
        row = pl.ds(8 * s, 8)
        acc_ref[row, :] = acc_ref[row, :] + g

    @pl.when(c == NC - 1)
    def _():
        acc = acc_ref[...]
        m = jnp.min(acc, axis=1)
        iota = lax.broadcasted_iota(jnp.int32, (BI, N_E), 1)
        hit = jnp.where(acc == m[:, None], iota, jnp.int32(N_E))
        j_ref[...] = jnp.min(hit, axis=1)

        s = jnp.sum(m)

        @pl.when(b == 0)
        def _():
            part_ref[0, 0] = 0.0

        part_ref[0, 0] += s


def _argmin_tc(xt, wt):
    return pl.pallas_call(
        _tc_body,
        grid=(NB, NC),
        in_specs=[
            pl.BlockSpec((8, BI), lambda b, c: (c, b)),
            pl.BlockSpec((8, N_E), lambda b, c: (c, 0)),
        ],
        out_specs=[
            pl.BlockSpec((BI,), lambda b, c: (b,)),
            pl.BlockSpec(memory_space=pltpu.SMEM, block_shape=(1, 1),
                         index_map=lambda b, c: (0, 0)),
        ],
        out_shape=[
            jax.ShapeDtypeStruct((R_TC,), jnp.int32),
            jax.ShapeDtypeStruct((1, 1), jnp.float32),
        ],
        scratch_shapes=[pltpu.VMEM((BI, N_E), jnp.float32)],
    )(xt, wt)


def _argmin_sc(x, wt):
    info = plsc.get_sparse_core_info()
    ncores = info.num_cores
    mesh = plsc.VectorSubcoreMesh(core_axis_name="c", subcore_axis_name="s")

    @functools.partial(
        pl.kernel,
        mesh=mesh,
        out_type=[
            jax.ShapeDtypeStruct((NS_ROWS, 16), jnp.float32),
            jax.ShapeDtypeStruct((NS_ROWS, 16), jnp.int32),
        ],
        scratch_types=[
            pltpu.VMEM((RPT, E_DIM), jnp.float32),
            pltpu.VMEM((E_DIM, CPP), jnp.float32),
            pltpu.VMEM((RPT, 16), jnp.float32),
            pltpu.VMEM((RPT, 16), jnp.int32),
            pltpu.SemaphoreType.DMA,
            pltpu.SemaphoreType.DMA,
        ],
    )
    def dist_k(x_hbm, wt_hbm, rm_hbm, ri_hbm,
               x_vm, wt_vm, rm_vm, ri_vm, semx, semw):
        wid = lax.axis_index("s") * ncores + lax.axis_index("c")
        base = wid * RPT
        pltpu.async_copy(x_hbm.at[pl.ds(R_TC + base, RPT)], x_vm, semx).wait()

        iota = lax.iota(jnp.int32, 16)

        def init_row(r, _):
            rm_vm[r, :] = jnp.full((16,), jnp.float32(jnp.inf))
            ri_vm[r, :] = jnp.full((16,), jnp.int32(N_E))
            return 0

        lax.fori_loop(0, RPT, init_row, 0)

        def do_pass(p, _):
            pltpu.async_copy(
                wt_hbm.at[:, pl.ds(p * CPP, CPP)], wt_vm, semw).wait()

            def do_rowpair(rp, _):
                r0 = 2 * rp
                acc0 = tuple(jnp.zeros((16,), jnp.float32)
                             for _ in range(NJG))
                acc1 = tuple(jnp.zeros((16,), jnp.float32)
                             for _ in range(NJG))

                def chunk2(cc, accs01):
                    a0, a1 = list(accs01[0]), list(accs01[1])
                    xv0 = x_vm[r0, pl.ds(16 * cc, 16)]
                    xv1 = x_vm[r0 + 1, pl.ds(16 * cc, 16)]
                    for half in range(2):
                        x0 = [jnp.broadcast_to(xv0[8 * half + k], (16,))
                              for k in range(8)]
                        x1 = [jnp.broadcast_to(xv1[8 * half + k], (16,))
                              for k in range(8)]
                        for jg in range(NJG):
                            wv = [wt_vm[16 * cc + 8 * half + k,
                                        pl.ds(16 * jg, 16)]
                                  for k in range(8)]

                            def tree(xb):
                                def sq(k):
                                    d = wv[k] - xb[k]
                                    return d * d
                                return ((sq(0) + sq(4)) + (sq(2) + sq(6))) + \
                                       ((sq(1) + sq(5)) + (sq(3) + sq(7)))

                            a0[jg] = a0[jg] + tree(x0)
                            a1[jg] = a1[jg] + tree(x1)
                    return (tuple(a0), tuple(a1))

                acc0, acc1 = lax.fori_loop(0, NC // 2, chunk2, (acc0, acc1))
                for i, accs in enumerate((acc0, acc1)):
                    r = r0 + i
                    rm = rm_vm[r, :]
                    ri = ri_vm[r, :]
                    for jg in range(NJG):
                        gidx = iota + (p * CPP + jg * 16)
                        lt = accs[jg] < rm
                        rm = jnp.where(lt, accs[jg], rm)
                        ri = jnp.where(lt, gidx, ri)
                    rm_vm[r, :] = rm
                    ri_vm[r, :] = ri
                return 0

            lax.fori_loop(0, RPT // 2, do_rowpair, 0)
            return 0

        lax.fori_loop(0, NPASS, do_pass, 0)

        pltpu.sync_copy(rm_vm, rm_hbm.at[pl.ds(base, RPT)])
        pltpu.sync_copy(ri_vm, ri_hbm.at[pl.ds(base, RPT)])

    return dist_k(x, wt)


def _finish_body(part_ref, rm_ref, ri_ref, j_ref, loss_ref):
    rm = rm_ref[...]
    mv = jnp.min(rm, axis=1)
    cand = jnp.where(rm == mv[:, None], ri_ref[...], jnp.int32(N_E))
    j_ref[...] = jnp.min(cand, axis=1)
    loss_ref[0, 0] = (part_ref[0, 0] + jnp.sum(mv)) * ((1.0 + ALPHA) / N_TOK)


def _finish(part, rm, ri):
    return pl.pallas_call(
        _finish_body,
        in_specs=[
            pl.BlockSpec(memory_space=pltpu.SMEM),
            pl.BlockSpec(memory_space=pltpu.VMEM),
            pl.BlockSpec(memory_space=pltpu.VMEM),
        ],
        out_specs=[
            pl.BlockSpec(memory_space=pltpu.VMEM),
            pl.BlockSpec(memory_space=pltpu.SMEM),
        ],
        out_shape=[
            jax.ShapeDtypeStruct((NS_ROWS,), jnp.int32),
            jax.ShapeDtypeStruct((1, 1), jnp.float32),
        ],
    )(part, rm, ri)


def _sc_gather(W, j):
    info = plsc.get_sparse_core_info()
    ncores, nsub = info.num_cores, info.num_subcores
    nw = ncores * nsub
    bpw = N_TOK // nw
    mesh = plsc.VectorSubcoreMesh(core_axis_name="c", subcore_axis_name="s")

    @functools.partial(
        pl.kernel,
        mesh=mesh,
        out_type=jax.ShapeDtypeStruct((N_TOK, E_DIM), jnp.float32),
        scratch_types=[
            pltpu.VMEM((bpw,), jnp.int32),
            pltpu.VMEM((bpw, E_DIM), jnp.float32),
            pltpu.SemaphoreType.DMA,
        ],
    )
    def gather_k(w_hbm, idx_hbm, out_hbm, idx_v, rows_v, sem):
        wid = lax.axis_index("s") * ncores + lax.axis_index("c")
        base = wid * bpw
        pltpu.sync_copy(idx_hbm.at[pl.ds(base, bpw)], idx_v)
        pltpu.async_copy(w_hbm.at[idx_v], rows_v, sem).wait()
        pltpu.sync_copy(rows_v, out_hbm.at[pl.ds(base, bpw)])

    return gather_k(W, j)


def kernel(x, W):
    xt = x.T
    wt = W.T
    rm, ri = _argmin_sc(x, wt)
    j_tc, part = _argmin_tc(xt, wt)
    j_sc, loss = _finish(part, rm, ri)
    j = jnp.concatenate([j_tc, j_sc])
    W_j = _sc_gather(W, j)
    return (W_j, loss.reshape(()))


# speedup vs baseline: 2.3005x; 2.3005x over previous
import functools

import jax
import jax.numpy as jnp
from jax import lax
from jax.experimental import pallas as pl
from jax.experimental.pallas import tpu as pltpu
from jax.experimental.pallas import tpu_sc as plsc

N_TOK = 2048
N_E = 1024
E_DIM = 256
ALPHA = 0.9

K_CAND = 8
BI = 256
NBI = N_TOK // BI
NC = E_DIM // 8


def _score_body(x_ref, wt_ref, cand_ref):
    xb = x_ref[...]
    wtb = wt_ref[...]
    wn = jnp.sum(wtb * wtb, axis=0)
    mm = lax.dot_general(xb, wtb, (((1,), (0,)), ((), ())),
                         preferred_element_type=jnp.float32,
                         precision=lax.Precision.HIGHEST)
    score = wn[None, :] - (mm + mm)
    iota = lax.broadcasted_iota(jnp.int32, (BI, N_E), 1)
    cols = []
    for _ in range(K_CAND):
        m = jnp.min(score, axis=1)
        hit = jnp.where(score == m[:, None], iota, jnp.int32(N_E))
        idx = jnp.min(hit, axis=1)
        cols.append(idx[:, None])
        score = jnp.where(iota == idx[:, None], jnp.float32(jnp.inf), score)
    cand_ref[...] = jnp.concatenate(cols, axis=1)


def _topk_scores(x, wt):
    return pl.pallas_call(
        _score_body,
        grid=(NBI,),
        in_specs=[
            pl.BlockSpec((BI, E_DIM), lambda b: (b, 0)),
            pl.BlockSpec((E_DIM, N_E), lambda b: (0, 0)),
        ],
        out_specs=pl.BlockSpec((BI, K_CAND), lambda b: (b, 0)),
        out_shape=jax.ShapeDtypeStruct((N_TOK, K_CAND), jnp.int32),
    )(x, wt)


def _tree_body(xt_ref, wgt_ref, candt_ref, j_ref, loss_ref, dis_ref):
    k = pl.program_id(0)
    xb = xt_ref[...]
    wb = wgt_ref[...]

    acc = jnp.zeros((8, N_TOK), jnp.float32)
    for c in range(NC):
        row = slice(8 * c, 8 * c + 8)
        d = wb[row, :] - xb[row, :]
        t = d * d
        u = t + jnp.roll(t, 4, axis=0)
        v = u + jnp.roll(u, 2, axis=0)
        w = v + jnp.roll(v, 1, axis=0)
        acc = acc + w
    dis_ref[pl.ds(k, 1), :] = acc[0:1, :]

    @pl.when(k == K_CAND - 1)
    def _():
        dis = dis_ref[...]
        mv = jnp.min(dis, axis=0)
        hit = jnp.where(dis == mv[None, :], candt_ref[...], jnp.int32(N_E))
        j_ref[...] = jnp.min(hit, axis=0)
        loss_ref[0, 0] = jnp.sum(mv) * ((1.0 + ALPHA) / N_TOK)


def _tree_argmin(xt, wgt, candt):
    return pl.pallas_call(
        _tree_body,
        grid=(K_CAND,),
        in_specs=[
            pl.BlockSpec((E_DIM, N_TOK), lambda k: (0, 0)),
            pl.BlockSpec((E_DIM, N_TOK), lambda k: (0, k)),
            pl.BlockSpec((K_CAND, N_TOK), lambda k: (0, 0)),
        ],
        out_specs=[
            pl.BlockSpec((N_TOK,), lambda k: (0,)),
            pl.BlockSpec(memory_space=pltpu.SMEM, block_shape=(1, 1),
                         index_map=lambda k: (0, 0)),
        ],
        out_shape=[
            jax.ShapeDtypeStruct((N_TOK,), jnp.int32),
            jax.ShapeDtypeStruct((1, 1), jnp.float32),
        ],
        scratch_shapes=[pltpu.VMEM((K_CAND, N_TOK), jnp.float32)],
    )(xt, wgt, candt)


def _sc_gather(W, j, rows_per_chunk=64):
    B = j.shape[0]
    info = plsc.get_sparse_core_info()
    ncores, nsub = info.num_cores, info.num_subcores
    nw = ncores * nsub
    bpw = B // nw
    nch = bpw // rows_per_chunk
    mesh = plsc.VectorSubcoreMesh(core_axis_name="c", subcore_axis_name="s")

    @functools.partial(
        pl.kernel,
        mesh=mesh,
        out_type=jax.ShapeDtypeStruct((B, E_DIM), jnp.float32),
        scratch_types=[
            pltpu.VMEM((bpw,), jnp.int32),
            pltpu.VMEM((rows_per_chunk, E_DIM), jnp.float32),
            pltpu.SemaphoreType.DMA,
        ],
    )
    def gather_k(w_hbm, idx_hbm, out_hbm, idx_v, rows_v, sem):
        wid = lax.axis_index("s") * ncores + lax.axis_index("c")
        base = wid * bpw
        pltpu.sync_copy(idx_hbm.at[pl.ds(base, bpw)], idx_v)
        for h in range(nch):
            pltpu.async_copy(
                w_hbm.at[idx_v.at[pl.ds(h * rows_per_chunk,
                                        rows_per_chunk)]],
                rows_v, sem).wait()
            pltpu.sync_copy(
                rows_v,
                out_hbm.at[pl.ds(base + h * rows_per_chunk,
                                 rows_per_chunk)])

    return gather_k(W, j)


def kernel(x, W):
    xt = x.T
    wt = W.T
    cand = _topk_scores(x, wt)
    candt = cand.T
    wg = _sc_gather(W, candt.reshape(-1), rows_per_chunk=256)
    wgt = wg.T
    j, loss = _tree_argmin(xt, wgt, candt)
    W_j = _sc_gather(W, j, rows_per_chunk=64)
    return (W_j, loss.reshape(()))


# speedup vs baseline: 2.4271x; 1.0550x over previous
import functools

import jax
import jax.numpy as jnp
from jax import lax
from jax.experimental import pallas as pl
from jax.experimental.pallas import tpu as pltpu
from jax.experimental.pallas import tpu_sc as plsc

N_TOK = 2048
N_E = 1024
E_DIM = 256
ALPHA = 0.9

K_CAND = 8
BI = 256
NBI = N_TOK // BI
NC = E_DIM // 8


def _score_body(x_ref, wt_ref, cand_ref, wn_ref):
    b = pl.program_id(0)
    xb = x_ref[...]
    wtb = wt_ref[...]

    @pl.when(b == 0)
    def _():
        wn_ref[...] = jnp.sum(wtb * wtb, axis=0, keepdims=True)

    mm = lax.dot_general(xb, wtb, (((1,), (0,)), ((), ())),
                         preferred_element_type=jnp.float32,
                         precision=lax.Precision.HIGHEST)
    score = wn_ref[...] - (mm + mm)
    iota = lax.broadcasted_iota(jnp.int32, (BI, N_E), 1)
    keys = (lax.bitcast_convert_type(score + 1.0, jnp.int32) &
            jnp.int32(~1023)) | iota
    cols = []
    for _ in range(K_CAND):
        m = jnp.min(keys, axis=1)
        cols.append((m & jnp.int32(1023))[:, None])
        keys = jnp.where(keys == m[:, None], jnp.int32(2**31 - 1), keys)
    cand_ref[...] = jnp.concatenate(cols, axis=1)


def _topk_scores(x, wt):
    return pl.pallas_call(
        _score_body,
        grid=(NBI,),
        in_specs=[
            pl.BlockSpec((BI, E_DIM), lambda b: (b, 0)),
            pl.BlockSpec((E_DIM, N_E), lambda b: (0, 0)),
        ],
        out_specs=pl.BlockSpec((BI, K_CAND), lambda b: (b, 0)),
        out_shape=jax.ShapeDtypeStruct((N_TOK, K_CAND), jnp.int32),
        scratch_shapes=[pltpu.VMEM((1, N_E), jnp.float32)],
    )(x, wt)


def _tree_body(xt_ref, wgt_ref, candt_ref, j_ref, loss_ref, dis_ref):
    k = pl.program_id(0)
    xb = xt_ref[...]
    wb = wgt_ref[...]

    acc = jnp.zeros((8, N_TOK), jnp.float32)
    for c in range(NC):
        row = slice(8 * c, 8 * c + 8)
        d = wb[row, :] - xb[row, :]
        t = d * d
        u = t + jnp.roll(t, 4, axis=0)
        v = u + jnp.roll(u, 2, axis=0)
        w = v + jnp.roll(v, 1, axis=0)
        acc = acc + w
    dis_ref[pl.ds(k, 1), :] = acc[0:1, :]

    @pl.when(k == K_CAND - 1)
    def _():
        dis = dis_ref[...]
        mv = jnp.min(dis, axis=0)
        hit = jnp.where(dis == mv[None, :], candt_ref[...], jnp.int32(N_E))
        j_ref[...] = jnp.min(hit, axis=0)
        loss_ref[0, 0] = jnp.sum(mv) * ((1.0 + ALPHA) / N_TOK)


def _tree_argmin(xt, wgt, candt):
    return pl.pallas_call(
        _tree_body,
        grid=(K_CAND,),
        in_specs=[
            pl.BlockSpec((E_DIM, N_TOK), lambda k: (0, 0)),
            pl.BlockSpec((E_DIM, N_TOK), lambda k: (0, k)),
            pl.BlockSpec((K_CAND, N_TOK), lambda k: (0, 0)),
        ],
        out_specs=[
            pl.BlockSpec((N_TOK,), lambda k: (0,)),
            pl.BlockSpec(memory_space=pltpu.SMEM, block_shape=(1, 1),
                         index_map=lambda k: (0, 0)),
        ],
        out_shape=[
            jax.ShapeDtypeStruct((N_TOK,), jnp.int32),
            jax.ShapeDtypeStruct((1, 1), jnp.float32),
        ],
        scratch_shapes=[pltpu.VMEM((K_CAND, N_TOK), jnp.float32)],
    )(xt, wgt, candt)


def _sc_gather(W, j, rows_per_chunk=64):
    B = j.shape[0]
    info = plsc.get_sparse_core_info()
    ncores, nsub = info.num_cores, info.num_subcores
    nw = ncores * nsub
    bpw = B // nw
    nch = bpw // rows_per_chunk
    mesh = plsc.VectorSubcoreMesh(core_axis_name="c", subcore_axis_name="s")

    @functools.partial(
        pl.kernel,
        mesh=mesh,
        out_type=jax.ShapeDtypeStruct((B, E_DIM), jnp.float32),
        scratch_types=[
            pltpu.VMEM((bpw,), jnp.int32),
            pltpu.VMEM((rows_per_chunk, E_DIM), jnp.float32),
            pltpu.SemaphoreType.DMA,
        ],
    )
    def gather_k(w_hbm, idx_hbm, out_hbm, idx_v, rows_v, sem):
        wid = lax.axis_index("s") * ncores + lax.axis_index("c")
        base = wid * bpw
        pltpu.sync_copy(idx_hbm.at[pl.ds(base, bpw)], idx_v)
        for h in range(nch):
            pltpu.async_copy(
                w_hbm.at[idx_v.at[pl.ds(h * rows_per_chunk,
                                        rows_per_chunk)]],
                rows_v, sem).wait()
            pltpu.sync_copy(
                rows_v,
                out_hbm.at[pl.ds(base + h * rows_per_chunk,
                                 rows_per_chunk)])

    return gather_k(W, j)


def kernel(x, W):
    xt = x.T
    wt = W.T
    cand = _topk_scores(x, wt)
    candt = cand.T
    wg = _sc_gather(W, candt.reshape(-1), rows_per_chunk=256)
    wgt = wg.T
    j, loss = _tree_argmin(xt, wgt, candt)
    W_j = _sc_gather(W, j, rows_per_chunk=64)
    return (W_j, loss.reshape(()))
